# baseline (speedup 1.0000x reference)
import functools

import jax
import jax.numpy as jnp
import numpy as np
from jax.experimental import pallas as pl
from jax.experimental.pallas import tpu as pltpu

_EPS = 1e-5


def _pixel_shuffle_3d(x, scale):
    B, C, D, H, W = x.shape
    n_out = C // scale ** 3
    x = x.reshape(B, n_out, scale, scale, scale, D, H, W)
    x = jnp.transpose(x, (0, 1, 5, 2, 6, 3, 7, 4))
    return x.reshape(B, n_out, D * scale, H * scale, W * scale)


@functools.lru_cache(maxsize=None)
def _hw_masks_np(D, H, W):
    r = np.arange(D * H * W)
    h = (r // W) % H
    w = r % W
    m = np.zeros((9, D * H * W), np.float32)
    j = 0
    for oh in (-1, 0, 1):
        for ow in (-1, 0, 1):
            valid = ((h + oh >= 0) & (h + oh < H) &
                     (w + ow >= 0) & (w + ow < W))
            m[j] = valid.astype(np.float32)
            j += 1
    return m


def _roll_lanes(x, k):
    L = x.shape[-1]
    k %= L
    if k == 0:
        return x
    return jnp.concatenate([x[:, k:], x[:, :k]], axis=1)


def _conv_core(xs, wg_ref, b_ref, alpha, m_ref, W, HW, add_identity):
    nb = len(xs)
    L = xs[0].shape[-1]
    chunks = []
    j = 0
    for oh in (-1, 0, 1):
        for ow in (-1, 0, 1):
            row = []
            for x in xs:
                xr = _roll_lanes(x, oh * W + ow)
                if not (oh == 0 and ow == 0):
                    xr = xr * m_ref[j]
                row.append(xr)
            chunks.append(row[0] if nb == 1 else
                          jnp.concatenate(row, axis=1))
            j += 1
    S = jnp.concatenate(chunks, axis=0)
    p_lo = jnp.dot(wg_ref[0], S, preferred_element_type=jnp.float32)
    p_mid = jnp.dot(wg_ref[1], S, preferred_element_type=jnp.float32)
    p_hi = jnp.dot(wg_ref[2], S, preferred_element_type=jnp.float32)
    cout = p_mid.shape[0]
    z = jnp.zeros((cout, HW), jnp.float32)
    ys = []
    for i in range(nb):
        sl = slice(i * L, (i + 1) * L)
        pm, plo, phi = p_mid[:, sl], p_lo[:, sl], p_hi[:, sl]
        acc = pm
        acc = acc + jnp.concatenate([z, plo[:, :L - HW]], axis=1)
        acc = acc + jnp.concatenate([phi[:, HW:], z], axis=1)
        c = acc + b_ref[...]
        y = jnp.where(c > 0, c, alpha * c)
        if add_identity:
            y = y + c
        ys.append(y)
    return ys


def _c0_kernel(alpha_ref, x_ref, wg_ref, b_ref, m_ref,
               a_ref, sum_ref, ssq_ref, *, W, HW, nb):
    ys = _conv_core([x_ref[i] for i in range(nb)], wg_ref, b_ref,
                    alpha_ref[0], m_ref, W, HW, True)
    for i, y in enumerate(ys):
        a_ref[i] = y.astype(jnp.bfloat16)
        sum_ref[i] = jnp.sum(y, axis=1, keepdims=True)
        ssq_ref[i] = jnp.sum(y * y, axis=1, keepdims=True)


def _fused_kernel(alpha_ref, scale_ref, shift_ref, a_prev_ref, res_ref,
                  wg_ref, b_ref, m_ref,
                  cur_ref, a_ref, sum_ref, ssq_ref, *, W, HW, has_res, nb):
    xs = []
    for i in range(nb):
        cur = a_prev_ref[i] * scale_ref[...] + shift_ref[...]
        if has_res:
            cur = cur + res_ref[i]
        x16 = cur.astype(jnp.bfloat16)
        cur_ref[i] = x16
        xs.append(x16)
    ys = _conv_core(xs, wg_ref, b_ref, alpha_ref[0], m_ref, W, HW, False)
    for i, y in enumerate(ys):
        a_ref[i] = y.astype(jnp.bfloat16)
        sum_ref[i] = jnp.sum(y, axis=1, keepdims=True)
        ssq_ref[i] = jnp.sum(y * y, axis=1, keepdims=True)


def _final_kernel(scale_ref, shift_ref, a_ref, res_ref, out_ref):
    out_ref[...] = (a_ref[...] * scale_ref[...] + shift_ref[...]
                    + res_ref[...])


def _prep_w(w, dtype):
    _, cout, cin = w.shape
    return (w.reshape(3, 9, cout, cin).transpose(0, 2, 1, 3)
            .reshape(3, cout, 9 * cin).astype(dtype))


def _bn_scale_shift(s1, s2, gamma, beta, n, add_self):
    s1 = jnp.sum(s1, axis=0).reshape(-1)
    s2 = jnp.sum(s2, axis=0).reshape(-1)
    mean = s1 / n
    var = jnp.maximum(s2 / n - mean * mean, 0.0)
    inv = gamma * jax.lax.rsqrt(var + _EPS)
    shift = beta - mean * inv
    scale = inv + (1.0 if add_self else 0.0)
    C = scale.shape[0]
    return scale.reshape(C, 1), shift.reshape(C, 1)


def _cparams():
    return pltpu.CompilerParams(
        dimension_semantics=("parallel",),
        vmem_limit_bytes=48 * 1024 * 1024)


def _conv0_call(alpha, x_flat, w, b, m32, *, W, HW, nb):
    B, cin, L = x_flat.shape
    cout = w.shape[1]
    wg = _prep_w(w, jnp.float32)
    out_shape = (
        jax.ShapeDtypeStruct((B, cout, L), jnp.bfloat16),
        jax.ShapeDtypeStruct((B, cout, 1), jnp.float32),
        jax.ShapeDtypeStruct((B, cout, 1), jnp.float32),
    )
    return pl.pallas_call(
        functools.partial(_c0_kernel, W=W, HW=HW, nb=nb),
        out_shape=out_shape,
        grid=(B // nb,),
        in_specs=[
            pl.BlockSpec(memory_space=pltpu.MemorySpace.SMEM),
            pl.BlockSpec((nb, cin, L), lambda b: (b, 0, 0)),
            pl.BlockSpec((3, cout, 9 * cin), lambda b: (0, 0, 0)),
            pl.BlockSpec((cout, 1), lambda b: (0, 0)),
            pl.BlockSpec((9, 1, L), lambda b: (0, 0, 0)),
        ],
        out_specs=(
            pl.BlockSpec((nb, cout, L), lambda b: (b, 0, 0)),
            pl.BlockSpec((nb, cout, 1), lambda b: (b, 0, 0)),
            pl.BlockSpec((nb, cout, 1), lambda b: (b, 0, 0)),
        ),
        compiler_params=_cparams(),
    )(alpha, x_flat, wg, b.reshape(cout, 1), m32)


def _fused_call(alpha, scale, shift, a_prev, res, w, b, m16, *, W, HW, nb):
    B, cin, L = a_prev.shape
    cout = w.shape[1]
    wg = _prep_w(w, jnp.bfloat16)
    has_res = res is not None
    bspec = pl.BlockSpec((nb, cin, L), lambda b: (b, 0, 0))
    cspec = pl.BlockSpec((cout, 1), lambda b: (0, 0))
    in_specs = [
        pl.BlockSpec(memory_space=pltpu.MemorySpace.SMEM),
        cspec, cspec,
        bspec,
    ]
    args = [alpha, scale, shift, a_prev]
    if has_res:
        in_specs.append(bspec)
        args.append(res)
    in_specs += [
        pl.BlockSpec((3, cout, 9 * cin), lambda b: (0, 0, 0)),
        cspec,
        pl.BlockSpec((9, 1, L), lambda b: (0, 0, 0)),
    ]
    args += [wg, b.reshape(cout, 1), m16]

    def body(alpha_ref, scale_ref, shift_ref, a_prev_ref, *rest):
        if has_res:
            res_ref = rest[0]
            rest = rest[1:]
        else:
            res_ref = None
        wg_ref, b_ref, m_ref, cur_ref, a_ref, sum_ref, ssq_ref = rest
        _fused_kernel(alpha_ref, scale_ref, shift_ref, a_prev_ref, res_ref,
                      wg_ref, b_ref, m_ref, cur_ref, a_ref, sum_ref, ssq_ref,
                      W=W, HW=HW, has_res=has_res, nb=nb)

    out_shape = (
        jax.ShapeDtypeStruct((B, cout, L), jnp.bfloat16),
        jax.ShapeDtypeStruct((B, cout, L), jnp.bfloat16),
        jax.ShapeDtypeStruct((B, cout, 1), jnp.float32),
        jax.ShapeDtypeStruct((B, cout, 1), jnp.float32),
    )
    return pl.pallas_call(
        body,
        out_shape=out_shape,
        grid=(B // nb,),
        in_specs=in_specs,
        out_specs=(
            pl.BlockSpec((nb, cout, L), lambda b: (b, 0, 0)),
            pl.BlockSpec((nb, cout, L), lambda b: (b, 0, 0)),
            pl.BlockSpec((nb, cout, 1), lambda b: (b, 0, 0)),
            pl.BlockSpec((nb, cout, 1), lambda b: (b, 0, 0)),
        ),
        compiler_params=_cparams(),
    )(*args)


def _final_call(scale, shift, a, res, *, nb):
    B, cout, L = a.shape
    bspec = pl.BlockSpec((nb, cout, L), lambda b: (b, 0, 0))
    cspec = pl.BlockSpec((cout, 1), lambda b: (0, 0))
    return pl.pallas_call(
        _final_kernel,
        out_shape=jax.ShapeDtypeStruct((B, cout, L), jnp.float32),
        grid=(B // nb,),
        in_specs=[cspec, cspec, bspec, bspec],
        out_specs=bspec,
        compiler_params=_cparams(),
    )(scale, shift, a, res)


def kernel(img,
           w0, b0, alpha0, gamma0, beta0,
           w1, b1, alpha1, gamma1, beta1,
           w2, b2, alpha2, gamma2, beta2,
           w3, b3, alpha3, gamma3, beta3):
    x = _pixel_shuffle_3d(img, 2)
    B, C0, D, H, W = x.shape
    L = D * H * W
    HW = H * W
    x_flat = x.reshape(B, C0, L)
    n = B * L

    m32 = jnp.asarray(_hw_masks_np(D, H, W)).reshape(9, 1, L)
    m16 = m32.astype(jnp.bfloat16)

    nbc = 2 if B % 2 == 0 else 1
    nbf = 8 if B % 8 == 0 else nbc
    a, s1, s2 = _conv0_call(alpha0, x_flat, w0, b0, m32, W=W, HW=HW, nb=nbc)
    sc, sh = _bn_scale_shift(s1, s2, gamma0, beta0, n, True)

    res = None
    for (w, b, alpha, gamma, beta) in (
            (w1, b1, alpha1, gamma1, beta1),
            (w2, b2, alpha2, gamma2, beta2),
            (w3, b3, alpha3, gamma3, beta3)):
        cur, a_new, s1, s2 = _fused_call(alpha, sc, sh, a, res, w, b, m16,
                                         W=W, HW=HW, nb=nbc)
        sc, sh = _bn_scale_shift(s1, s2, gamma, beta, n, False)
        a, res = a_new, cur

    out = _final_call(sc, sh, a, res, nb=nbf)
    cout = out.shape[1]
    return out.reshape(B, cout, D, H, W)


# speedup vs baseline: 1.1618x; 1.1618x over previous
import functools

import jax
import jax.numpy as jnp
import numpy as np
from jax.experimental import pallas as pl
from jax.experimental.pallas import tpu as pltpu

_EPS = 1e-5


def _pixel_shuffle_3d(x, scale):
    B, C, D, H, W = x.shape
    n_out = C // scale ** 3
    x = x.reshape(B, n_out, scale, scale, scale, D, H, W)
    x = jnp.transpose(x, (0, 1, 5, 2, 6, 3, 7, 4))
    return x.reshape(B, n_out, D * scale, H * scale, W * scale)


@functools.lru_cache(maxsize=None)
def _hw_masks_np(D, H, W):
    r = np.arange(D * H * W)
    h = (r // W) % H
    w = r % W
    m = np.zeros((9, D * H * W), np.float32)
    j = 0
    for oh in (-1, 0, 1):
        for ow in (-1, 0, 1):
            valid = ((h + oh >= 0) & (h + oh < H) &
                     (w + ow >= 0) & (w + ow < W))
            m[j] = valid.astype(np.float32)
            j += 1
    return m


def _roll_lanes(x, k):
    L = x.shape[-1]
    k %= L
    if k == 0:
        return x
    return jnp.concatenate([x[:, k:], x[:, :k]], axis=1)


def _conv_core(xs, wg_ref, b_ref, alpha, m_ref, W, HW, add_identity):
    nb = len(xs)
    L = xs[0].shape[-1]
    chunks = []
    j = 0
    for oh in (-1, 0, 1):
        for ow in (-1, 0, 1):
            row = []
            for x in xs:
                xr = _roll_lanes(x, oh * W + ow)
                if not (oh == 0 and ow == 0):
                    xr = xr * m_ref[j]
                row.append(xr)
            chunks.append(row[0] if nb == 1 else
                          jnp.concatenate(row, axis=1))
            j += 1
    S = jnp.concatenate(chunks, axis=0)
    p_lo = jnp.dot(wg_ref[0], S, preferred_element_type=jnp.float32)
    p_mid = jnp.dot(wg_ref[1], S, preferred_element_type=jnp.float32)
    p_hi = jnp.dot(wg_ref[2], S, preferred_element_type=jnp.float32)
    cout = p_mid.shape[0]
    z = jnp.zeros((cout, HW), jnp.float32)
    ys = []
    for i in range(nb):
        sl = slice(i * L, (i + 1) * L)
        pm, plo, phi = p_mid[:, sl], p_lo[:, sl], p_hi[:, sl]
        acc = pm
        acc = acc + jnp.concatenate([z, plo[:, :L - HW]], axis=1)
        acc = acc + jnp.concatenate([phi[:, HW:], z], axis=1)
        c = acc + b_ref[...]
        y = jnp.where(c > 0, c, alpha * c)
        if add_identity:
            y = y + c
        ys.append(y)
    return ys


def _c0_kernel(alpha_ref, x_ref, wg_ref, b_ref, m_ref,
               a_ref, sum_ref, ssq_ref, *, W, HW, nb):
    for i in range(nb):
        (y,) = _conv_core([x_ref[i]], wg_ref, b_ref,
                          alpha_ref[0], m_ref, W, HW, True)
        a_ref[i] = y.astype(jnp.bfloat16)
        sum_ref[i] = jnp.sum(y, axis=1, keepdims=True)
        ssq_ref[i] = jnp.sum(y * y, axis=1, keepdims=True)


def _fused_kernel(alpha_ref, scale_ref, shift_ref, a_prev_ref, res_ref,
                  wg_ref, b_ref, m_ref,
                  cur_ref, a_ref, sum_ref, ssq_ref, *, W, HW, has_res, nb):
    for i in range(nb):
        cur = a_prev_ref[i] * scale_ref[...] + shift_ref[...]
        if has_res:
            cur = cur + res_ref[i]
        x16 = cur.astype(jnp.bfloat16)
        cur_ref[i] = x16
        (y,) = _conv_core([x16], wg_ref, b_ref, alpha_ref[0], m_ref,
                          W, HW, False)
        a_ref[i] = y.astype(jnp.bfloat16)
        sum_ref[i] = jnp.sum(y, axis=1, keepdims=True)
        ssq_ref[i] = jnp.sum(y * y, axis=1, keepdims=True)


def _final_kernel(scale_ref, shift_ref, a_ref, res_ref, out_ref):
    out_ref[...] = (a_ref[...] * scale_ref[...] + shift_ref[...]
                    + res_ref[...])


def _prep_w(w, dtype):
    _, cout, cin = w.shape
    return (w.reshape(3, 9, cout, cin).transpose(0, 2, 1, 3)
            .reshape(3, cout, 9 * cin).astype(dtype))


def _bn_scale_shift(s1, s2, gamma, beta, n, add_self):
    s1 = jnp.sum(s1, axis=0).reshape(-1)
    s2 = jnp.sum(s2, axis=0).reshape(-1)
    mean = s1 / n
    var = jnp.maximum(s2 / n - mean * mean, 0.0)
    inv = gamma * jax.lax.rsqrt(var + _EPS)
    shift = beta - mean * inv
    scale = inv + (1.0 if add_self else 0.0)
    C = scale.shape[0]
    return scale.reshape(C, 1), shift.reshape(C, 1)


def _cparams():
    return pltpu.CompilerParams(
        dimension_semantics=("parallel",),
        vmem_limit_bytes=48 * 1024 * 1024)


def _conv0_call(alpha, x_flat, w, b, m32, *, W, HW, nb):
    B, cin, L = x_flat.shape
    cout = w.shape[1]
    wg = _prep_w(w, jnp.float32)
    out_shape = (
        jax.ShapeDtypeStruct((B, cout, L), jnp.bfloat16),
        jax.ShapeDtypeStruct((B, cout, 1), jnp.float32),
        jax.ShapeDtypeStruct((B, cout, 1), jnp.float32),
    )
    return pl.pallas_call(
        functools.partial(_c0_kernel, W=W, HW=HW, nb=nb),
        out_shape=out_shape,
        grid=(B // nb,),
        in_specs=[
            pl.BlockSpec(memory_space=pltpu.MemorySpace.SMEM),
            pl.BlockSpec((nb, cin, L), lambda b: (b, 0, 0)),
            pl.BlockSpec((3, cout, 9 * cin), lambda b: (0, 0, 0)),
            pl.BlockSpec((cout, 1), lambda b: (0, 0)),
            pl.BlockSpec((9, 1, L), lambda b: (0, 0, 0)),
        ],
        out_specs=(
            pl.BlockSpec((nb, cout, L), lambda b: (b, 0, 0)),
            pl.BlockSpec((nb, cout, 1), lambda b: (b, 0, 0)),
            pl.BlockSpec((nb, cout, 1), lambda b: (b, 0, 0)),
        ),
        compiler_params=_cparams(),
    )(alpha, x_flat, wg, b.reshape(cout, 1), m32)


def _fused_call(alpha, scale, shift, a_prev, res, w, b, m16, *, W, HW, nb):
    B, cin, L = a_prev.shape
    cout = w.shape[1]
    wg = _prep_w(w, jnp.bfloat16)
    has_res = res is not None
    bspec = pl.BlockSpec((nb, cin, L), lambda b: (b, 0, 0))
    cspec = pl.BlockSpec((cout, 1), lambda b: (0, 0))
    in_specs = [
        pl.BlockSpec(memory_space=pltpu.MemorySpace.SMEM),
        cspec, cspec,
        bspec,
    ]
    args = [alpha, scale, shift, a_prev]
    if has_res:
        in_specs.append(bspec)
        args.append(res)
    in_specs += [
        pl.BlockSpec((3, cout, 9 * cin), lambda b: (0, 0, 0)),
        cspec,
        pl.BlockSpec((9, 1, L), lambda b: (0, 0, 0)),
    ]
    args += [wg, b.reshape(cout, 1), m16]

    def body(alpha_ref, scale_ref, shift_ref, a_prev_ref, *rest):
        if has_res:
            res_ref = rest[0]
            rest = rest[1:]
        else:
            res_ref = None
        wg_ref, b_ref, m_ref, cur_ref, a_ref, sum_ref, ssq_ref = rest
        _fused_kernel(alpha_ref, scale_ref, shift_ref, a_prev_ref, res_ref,
                      wg_ref, b_ref, m_ref, cur_ref, a_ref, sum_ref, ssq_ref,
                      W=W, HW=HW, has_res=has_res, nb=nb)

    out_shape = (
        jax.ShapeDtypeStruct((B, cout, L), jnp.bfloat16),
        jax.ShapeDtypeStruct((B, cout, L), jnp.bfloat16),
        jax.ShapeDtypeStruct((B, cout, 1), jnp.float32),
        jax.ShapeDtypeStruct((B, cout, 1), jnp.float32),
    )
    return pl.pallas_call(
        body,
        out_shape=out_shape,
        grid=(B // nb,),
        in_specs=in_specs,
        out_specs=(
            pl.BlockSpec((nb, cout, L), lambda b: (b, 0, 0)),
            pl.BlockSpec((nb, cout, L), lambda b: (b, 0, 0)),
            pl.BlockSpec((nb, cout, 1), lambda b: (b, 0, 0)),
            pl.BlockSpec((nb, cout, 1), lambda b: (b, 0, 0)),
        ),
        compiler_params=_cparams(),
    )(*args)


def _final_call(scale, shift, a, res, *, nb):
    B, cout, L = a.shape
    bspec = pl.BlockSpec((nb, cout, L), lambda b: (b, 0, 0))
    cspec = pl.BlockSpec((cout, 1), lambda b: (0, 0))
    return pl.pallas_call(
        _final_kernel,
        out_shape=jax.ShapeDtypeStruct((B, cout, L), jnp.float32),
        grid=(B // nb,),
        in_specs=[cspec, cspec, bspec, bspec],
        out_specs=bspec,
        compiler_params=_cparams(),
    )(scale, shift, a, res)


def kernel(img,
           w0, b0, alpha0, gamma0, beta0,
           w1, b1, alpha1, gamma1, beta1,
           w2, b2, alpha2, gamma2, beta2,
           w3, b3, alpha3, gamma3, beta3):
    x = _pixel_shuffle_3d(img, 2)
    B, C0, D, H, W = x.shape
    L = D * H * W
    HW = H * W
    x_flat = x.reshape(B, C0, L)
    n = B * L

    m32 = jnp.asarray(_hw_masks_np(D, H, W)).reshape(9, 1, L)
    m16 = m32.astype(jnp.bfloat16)

    nbc = 4 if B % 4 == 0 else 1
    nbf = 8 if B % 8 == 0 else nbc
    a, s1, s2 = _conv0_call(alpha0, x_flat, w0, b0, m32, W=W, HW=HW, nb=nbc)
    sc, sh = _bn_scale_shift(s1, s2, gamma0, beta0, n, True)

    res = None
    for (w, b, alpha, gamma, beta) in (
            (w1, b1, alpha1, gamma1, beta1),
            (w2, b2, alpha2, gamma2, beta2),
            (w3, b3, alpha3, gamma3, beta3)):
        cur, a_new, s1, s2 = _fused_call(alpha, sc, sh, a, res, w, b, m16,
                                         W=W, HW=HW, nb=nbc)
        sc, sh = _bn_scale_shift(s1, s2, gamma, beta, n, False)
        a, res = a_new, cur

    out = _final_call(sc, sh, a, res, nb=nbf)
    cout = out.shape[1]
    return out.reshape(B, cout, D, H, W)


# speedup vs baseline: 1.2006x; 1.0334x over previous
import functools

import jax
import jax.numpy as jnp
import numpy as np
from jax.experimental import pallas as pl
from jax.experimental.pallas import tpu as pltpu

_EPS = 1e-5


def _pixel_shuffle_3d(x, scale):
    B, C, D, H, W = x.shape
    n_out = C // scale ** 3
    x = x.reshape(B, n_out, scale, scale, scale, D, H, W)
    x = jnp.transpose(x, (0, 1, 5, 2, 6, 3, 7, 4))
    return x.reshape(B, n_out, D * scale, H * scale, W * scale)


@functools.lru_cache(maxsize=None)
def _hw_masks_np(D, H, W):
    r = np.arange(D * H * W)
    h = (r // W) % H
    w = r % W
    m = np.zeros((9, D * H * W), np.float32)
    j = 0
    for oh in (-1, 0, 1):
        for ow in (-1, 0, 1):
            valid = ((h + oh >= 0) & (h + oh < H) &
                     (w + ow >= 0) & (w + ow < W))
            m[j] = valid.astype(np.float32)
            j += 1
    return m


def _roll_lanes(x, k):
    L = x.shape[-1]
    k %= L
    if k == 0:
        return x
    return jnp.concatenate([x[:, k:], x[:, :k]], axis=1)


def _conv_core(xs, wg_ref, b_ref, alpha, m_ref, W, HW, add_identity):
    nb = len(xs)
    L = xs[0].shape[-1]
    chunks = []
    j = 0
    for oh in (-1, 0, 1):
        for ow in (-1, 0, 1):
            row = []
            for x in xs:
                xr = _roll_lanes(x, oh * W + ow)
                if not (oh == 0 and ow == 0):
                    xr = xr * m_ref[j]
                row.append(xr)
            chunks.append(row[0] if nb == 1 else
                          jnp.concatenate(row, axis=1))
            j += 1
    S = jnp.concatenate(chunks, axis=0)
    p_lo = jnp.dot(wg_ref[0], S, preferred_element_type=jnp.float32)
    p_mid = jnp.dot(wg_ref[1], S, preferred_element_type=jnp.float32)
    p_hi = jnp.dot(wg_ref[2], S, preferred_element_type=jnp.float32)
    cout = p_mid.shape[0]
    z = jnp.zeros((cout, HW), jnp.float32)
    ys = []
    for i in range(nb):
        sl = slice(i * L, (i + 1) * L)
        pm, plo, phi = p_mid[:, sl], p_lo[:, sl], p_hi[:, sl]
        acc = pm
        acc = acc + jnp.concatenate([z, plo[:, :L - HW]], axis=1)
        acc = acc + jnp.concatenate([phi[:, HW:], z], axis=1)
        c = acc + b_ref[...]
        y = jnp.where(c > 0, c, alpha * c)
        if add_identity:
            y = y + c
        ys.append(y)
    return ys


def _bn_finalize(b, loc1, loc2, gamma_ref, beta_ref,
                 scale_ref, shift_ref, s1_sc, s2_sc, *,
                 steps, n, add_one):
    @pl.when(b == 0)
    def _init():
        s1_sc[...] = loc1
        s2_sc[...] = loc2

    @pl.when(b > 0)
    def _acc():
        s1_sc[...] += loc1
        s2_sc[...] += loc2

    @pl.when(b == steps - 1)
    def _fin():
        mean = s1_sc[...] / n
        var = jnp.maximum(s2_sc[...] / n - mean * mean, 0.0)
        inv = gamma_ref[...] * jax.lax.rsqrt(var + _EPS)
        shift_ref[...] = beta_ref[...] - mean * inv
        scale_ref[...] = inv + add_one


def _c0_kernel(alpha_ref, gamma_ref, beta_ref, x_ref, wg_ref, b_ref, m_ref,
               a_ref, scale_ref, shift_ref, s1_sc, s2_sc, *,
               W, HW, nb, steps, n, add_one):
    loc1 = loc2 = 0.0
    for i in range(nb):
        (y,) = _conv_core([x_ref[i]], wg_ref, b_ref,
                          alpha_ref[0], m_ref, W, HW, True)
        a_ref[i] = y.astype(jnp.bfloat16)
        loc1 = loc1 + jnp.sum(y, axis=1, keepdims=True)
        loc2 = loc2 + jnp.sum(y * y, axis=1, keepdims=True)
    _bn_finalize(pl.program_id(0), loc1, loc2, gamma_ref, beta_ref,
                 scale_ref, shift_ref, s1_sc, s2_sc,
                 steps=steps, n=n, add_one=add_one)


def _fused_kernel(alpha_ref, gamma_ref, beta_ref, scale_ref, shift_ref,
                  a_prev_ref, res_ref, wg_ref, b_ref, m_ref,
                  cur_ref, a_ref, oscale_ref, oshift_ref, s1_sc, s2_sc, *,
                  W, HW, has_res, nb, steps, n, add_one):
    loc1 = loc2 = 0.0
    for i in range(nb):
        cur = a_prev_ref[i] * scale_ref[...] + shift_ref[...]
        if has_res:
            cur = cur + res_ref[i]
        x16 = cur.astype(jnp.bfloat16)
        cur_ref[i] = x16
        (y,) = _conv_core([x16], wg_ref, b_ref, alpha_ref[0], m_ref,
                          W, HW, False)
        a_ref[i] = y.astype(jnp.bfloat16)
        loc1 = loc1 + jnp.sum(y, axis=1, keepdims=True)
        loc2 = loc2 + jnp.sum(y * y, axis=1, keepdims=True)
    _bn_finalize(pl.program_id(0), loc1, loc2, gamma_ref, beta_ref,
                 oscale_ref, oshift_ref, s1_sc, s2_sc,
                 steps=steps, n=n, add_one=add_one)


def _final_kernel(scale_ref, shift_ref, a_ref, res_ref, out_ref):
    out_ref[...] = (a_ref[...] * scale_ref[...] + shift_ref[...]
                    + res_ref[...])


def _prep_w(w, dtype):
    _, cout, cin = w.shape
    return (w.reshape(3, 9, cout, cin).transpose(0, 2, 1, 3)
            .reshape(3, cout, 9 * cin).astype(dtype))


def _cparams():
    return pltpu.CompilerParams(
        dimension_semantics=("arbitrary",),
        vmem_limit_bytes=48 * 1024 * 1024)


def _conv0_call(alpha, gamma, beta, x_flat, w, b, m32, *, W, HW, nb,
                add_self_next):
    B, cin, L = x_flat.shape
    cout = w.shape[1]
    wg = _prep_w(w, jnp.float32)
    steps = B // nb
    cspec = pl.BlockSpec((cout, 1), lambda b: (0, 0))
    out_shape = (
        jax.ShapeDtypeStruct((B, cout, L), jnp.bfloat16),
        jax.ShapeDtypeStruct((cout, 1), jnp.float32),
        jax.ShapeDtypeStruct((cout, 1), jnp.float32),
    )
    return pl.pallas_call(
        functools.partial(_c0_kernel, W=W, HW=HW, nb=nb, steps=steps,
                          n=float(B * L),
                          add_one=1.0 if add_self_next else 0.0),
        out_shape=out_shape,
        grid=(steps,),
        in_specs=[
            pl.BlockSpec(memory_space=pltpu.MemorySpace.SMEM),
            cspec, cspec,
            pl.BlockSpec((nb, cin, L), lambda b: (b, 0, 0)),
            pl.BlockSpec((3, cout, 9 * cin), lambda b: (0, 0, 0)),
            cspec,
            pl.BlockSpec((9, 1, L), lambda b: (0, 0, 0)),
        ],
        out_specs=(
            pl.BlockSpec((nb, cout, L), lambda b: (b, 0, 0)),
            cspec, cspec,
        ),
        scratch_shapes=[pltpu.VMEM((cout, 1), jnp.float32),
                        pltpu.VMEM((cout, 1), jnp.float32)],
        compiler_params=_cparams(),
    )(alpha, gamma.reshape(cout, 1), beta.reshape(cout, 1),
      x_flat, wg, b.reshape(cout, 1), m32)


def _fused_call(alpha, gamma, beta, scale, shift, a_prev, res, w, b, m16, *,
                W, HW, nb, add_self_next):
    B, cin, L = a_prev.shape
    cout = w.shape[1]
    wg = _prep_w(w, jnp.bfloat16)
    has_res = res is not None
    steps = B // nb
    bspec = pl.BlockSpec((nb, cin, L), lambda b: (b, 0, 0))
    cspec = pl.BlockSpec((cout, 1), lambda b: (0, 0))
    in_specs = [
        pl.BlockSpec(memory_space=pltpu.MemorySpace.SMEM),
        cspec, cspec,
        cspec, cspec,
        bspec,
    ]
    args = [alpha, gamma.reshape(cout, 1), beta.reshape(cout, 1),
            scale, shift, a_prev]
    if has_res:
        in_specs.append(bspec)
        args.append(res)
    in_specs += [
        pl.BlockSpec((3, cout, 9 * cin), lambda b: (0, 0, 0)),
        cspec,
        pl.BlockSpec((9, 1, L), lambda b: (0, 0, 0)),
    ]
    args += [wg, b.reshape(cout, 1), m16]

    def body(alpha_ref, gamma_ref, beta_ref, scale_ref, shift_ref,
             a_prev_ref, *rest):
        if has_res:
            res_ref = rest[0]
            rest = rest[1:]
        else:
            res_ref = None
        (wg_ref, b_ref, m_ref, cur_ref, a_ref, oscale_ref, oshift_ref,
         s1_sc, s2_sc) = rest
        _fused_kernel(alpha_ref, gamma_ref, beta_ref, scale_ref, shift_ref,
                      a_prev_ref, res_ref, wg_ref, b_ref, m_ref,
                      cur_ref, a_ref, oscale_ref, oshift_ref, s1_sc, s2_sc,
                      W=W, HW=HW, has_res=has_res, nb=nb, steps=steps,
                      n=float(B * L),
                      add_one=1.0 if add_self_next else 0.0)

    out_shape = (
        jax.ShapeDtypeStruct((B, cout, L), jnp.bfloat16),
        jax.ShapeDtypeStruct((B, cout, L), jnp.bfloat16),
        jax.ShapeDtypeStruct((cout, 1), jnp.float32),
        jax.ShapeDtypeStruct((cout, 1), jnp.float32),
    )
    return pl.pallas_call(
        body,
        out_shape=out_shape,
        grid=(steps,),
        in_specs=in_specs,
        out_specs=(
            pl.BlockSpec((nb, cout, L), lambda b: (b, 0, 0)),
            pl.BlockSpec((nb, cout, L), lambda b: (b, 0, 0)),
            cspec, cspec,
        ),
        scratch_shapes=[pltpu.VMEM((cout, 1), jnp.float32),
                        pltpu.VMEM((cout, 1), jnp.float32)],
        compiler_params=_cparams(),
    )(*args)


def _final_call(scale, shift, a, res, *, nb):
    B, cout, L = a.shape
    bspec = pl.BlockSpec((nb, cout, L), lambda b: (b, 0, 0))
    cspec = pl.BlockSpec((cout, 1), lambda b: (0, 0))
    return pl.pallas_call(
        _final_kernel,
        out_shape=jax.ShapeDtypeStruct((B, cout, L), jnp.float32),
        grid=(B // nb,),
        in_specs=[cspec, cspec, bspec, bspec],
        out_specs=bspec,
        compiler_params=_cparams(),
    )(scale, shift, a, res)


def kernel(img,
           w0, b0, alpha0, gamma0, beta0,
           w1, b1, alpha1, gamma1, beta1,
           w2, b2, alpha2, gamma2, beta2,
           w3, b3, alpha3, gamma3, beta3):
    x = _pixel_shuffle_3d(img, 2)
    B, C0, D, H, W = x.shape
    L = D * H * W
    HW = H * W
    x_flat = x.reshape(B, C0, L)

    m32 = jnp.asarray(_hw_masks_np(D, H, W)).reshape(9, 1, L)
    m16 = m32.astype(jnp.bfloat16)

    nbc = 4 if B % 4 == 0 else 1
    nbf = 8 if B % 8 == 0 else nbc
    a, sc, sh = _conv0_call(alpha0, gamma0, beta0, x_flat, w0, b0, m32,
                            W=W, HW=HW, nb=nbc, add_self_next=True)

    res = None
    for (w, b, alpha, gamma, beta) in (
            (w1, b1, alpha1, gamma1, beta1),
            (w2, b2, alpha2, gamma2, beta2),
            (w3, b3, alpha3, gamma3, beta3)):
        cur, a_new, sc, sh = _fused_call(alpha, gamma, beta, sc, sh, a, res,
                                         w, b, m16, W=W, HW=HW, nb=nbc,
                                         add_self_next=False)
        a, res = a_new, cur

    out = _final_call(sc, sh, a, res, nb=nbf)
    cout = out.shape[1]
    return out.reshape(B, cout, D, H, W)


# speedup vs baseline: 1.2318x; 1.0260x over previous
import functools

import jax
import jax.numpy as jnp
import numpy as np
from jax.experimental import pallas as pl
from jax.experimental.pallas import tpu as pltpu

_EPS = 1e-5


def _pixel_shuffle_3d(x, scale):
    B, C, D, H, W = x.shape
    n_out = C // scale ** 3
    x = x.reshape(B, n_out, scale, scale, scale, D, H, W)
    x = jnp.transpose(x, (0, 1, 5, 2, 6, 3, 7, 4))
    return x.reshape(B, n_out, D * scale, H * scale, W * scale)


@functools.lru_cache(maxsize=None)
def _hw_masks_np(D, H, W):
    r = np.arange(D * H * W)
    h = (r // W) % H
    w = r % W
    m = np.zeros((9, D * H * W), np.float32)
    j = 0
    for oh in (-1, 0, 1):
        for ow in (-1, 0, 1):
            valid = ((h + oh >= 0) & (h + oh < H) &
                     (w + ow >= 0) & (w + ow < W))
            m[j] = valid.astype(np.float32)
            j += 1
    return m


def _roll_lanes(x, k):
    L = x.shape[-1]
    k %= L
    if k == 0:
        return x
    return jnp.concatenate([x[:, k:], x[:, :k]], axis=1)


def _conv_core(xs, wg_ref, b_ref, alpha, m_ref, W, HW, add_identity):
    nb = len(xs)
    L = xs[0].shape[-1]
    chunks = []
    j = 0
    for oh in (-1, 0, 1):
        for ow in (-1, 0, 1):
            row = []
            for x in xs:
                xr = _roll_lanes(x, oh * W + ow)
                if not (oh == 0 and ow == 0):
                    xr = xr * m_ref[j]
                row.append(xr)
            chunks.append(row[0] if nb == 1 else
                          jnp.concatenate(row, axis=1))
            j += 1
    S = jnp.concatenate(chunks, axis=0)
    p_lo = jnp.dot(wg_ref[0], S, preferred_element_type=jnp.float32)
    p_mid = jnp.dot(wg_ref[1], S, preferred_element_type=jnp.float32)
    p_hi = jnp.dot(wg_ref[2], S, preferred_element_type=jnp.float32)
    cout = p_mid.shape[0]
    z = jnp.zeros((cout, HW), jnp.float32)
    ys = []
    for i in range(nb):
        sl = slice(i * L, (i + 1) * L)
        pm, plo, phi = p_mid[:, sl], p_lo[:, sl], p_hi[:, sl]
        acc = pm
        acc = acc + jnp.concatenate([z, plo[:, :L - HW]], axis=1)
        acc = acc + jnp.concatenate([phi[:, HW:], z], axis=1)
        c = acc + b_ref[...]
        y = jnp.where(c > 0, c, alpha * c)
        if add_identity:
            y = y + c
        ys.append(y)
    return ys


def _bn_finalize(b, loc1, loc2, gamma_ref, beta_ref,
                 scale_ref, shift_ref, s1_sc, s2_sc, *,
                 steps, n, add_one):
    @pl.when(b == 0)
    def _init():
        s1_sc[...] = loc1
        s2_sc[...] = loc2

    @pl.when(b > 0)
    def _acc():
        s1_sc[...] += loc1
        s2_sc[...] += loc2

    @pl.when(b == steps - 1)
    def _fin():
        mean = s1_sc[...] / n
        var = jnp.maximum(s2_sc[...] / n - mean * mean, 0.0)
        inv = gamma_ref[...] * jax.lax.rsqrt(var + _EPS)
        shift_ref[...] = beta_ref[...] - mean * inv
        scale_ref[...] = inv + add_one


def _c0_kernel(alpha_ref, gamma_ref, beta_ref, x_ref, wg_ref, b_ref, m_ref,
               a_ref, scale_ref, shift_ref, s1_sc, s2_sc, *,
               W, HW, nb, steps, n, add_one):
    loc1 = loc2 = 0.0
    for i in range(nb):
        (y,) = _conv_core([x_ref[i]], wg_ref, b_ref,
                          alpha_ref[0], m_ref, W, HW, True)
        a_ref[i] = y.astype(jnp.bfloat16)
        loc1 = loc1 + jnp.sum(y, axis=1, keepdims=True)
        loc2 = loc2 + jnp.sum(y * y, axis=1, keepdims=True)
    _bn_finalize(pl.program_id(0), loc1, loc2, gamma_ref, beta_ref,
                 scale_ref, shift_ref, s1_sc, s2_sc,
                 steps=steps, n=n, add_one=add_one)


def _fused_kernel(alpha_ref, gamma_ref, beta_ref, scale_ref, shift_ref,
                  a_prev_ref, res_ref, wg_ref, b_ref, m_ref,
                  cur_ref, a_ref, oscale_ref, oshift_ref, s1_sc, s2_sc, *,
                  W, HW, has_res, nb, steps, n, add_one):
    loc1 = loc2 = 0.0
    for i in range(nb):
        cur = a_prev_ref[i] * scale_ref[...] + shift_ref[...]
        if has_res:
            cur = cur + res_ref[i]
        x16 = cur.astype(jnp.bfloat16)
        cur_ref[i] = x16
        (y,) = _conv_core([x16], wg_ref, b_ref, alpha_ref[0], m_ref,
                          W, HW, False)
        a_ref[i] = y.astype(jnp.bfloat16)
        loc1 = loc1 + jnp.sum(y, axis=1, keepdims=True)
        loc2 = loc2 + jnp.sum(y * y, axis=1, keepdims=True)
    _bn_finalize(pl.program_id(0), loc1, loc2, gamma_ref, beta_ref,
                 oscale_ref, oshift_ref, s1_sc, s2_sc,
                 steps=steps, n=n, add_one=add_one)


def _final_kernel(scale_ref, shift_ref, a_ref, res_ref, out_ref):
    out_ref[...] = (a_ref[...] * scale_ref[...] + shift_ref[...]
                    + res_ref[...])


def _prep_w(w, dtype):
    _, cout, cin = w.shape
    return (w.reshape(3, 9, cout, cin).transpose(0, 2, 1, 3)
            .reshape(3, cout, 9 * cin).astype(dtype))


def _cparams():
    return pltpu.CompilerParams(
        dimension_semantics=("arbitrary",),
        vmem_limit_bytes=57 * 1024 * 1024)


def _conv0_call(alpha, gamma, beta, x_flat, w, b, m32, *, W, HW, nb,
                add_self_next):
    B, cin, L = x_flat.shape
    cout = w.shape[1]
    wg = _prep_w(w, jnp.float32)
    steps = B // nb
    cspec = pl.BlockSpec((cout, 1), lambda b: (0, 0))
    out_shape = (
        jax.ShapeDtypeStruct((B, cout, L), jnp.bfloat16),
        jax.ShapeDtypeStruct((cout, 1), jnp.float32),
        jax.ShapeDtypeStruct((cout, 1), jnp.float32),
    )
    return pl.pallas_call(
        functools.partial(_c0_kernel, W=W, HW=HW, nb=nb, steps=steps,
                          n=float(B * L),
                          add_one=1.0 if add_self_next else 0.0),
        out_shape=out_shape,
        grid=(steps,),
        in_specs=[
            pl.BlockSpec(memory_space=pltpu.MemorySpace.SMEM),
            cspec, cspec,
            pl.BlockSpec((nb, cin, L), lambda b: (b, 0, 0)),
            pl.BlockSpec((3, cout, 9 * cin), lambda b: (0, 0, 0)),
            cspec,
            pl.BlockSpec((9, 1, L), lambda b: (0, 0, 0)),
        ],
        out_specs=(
            pl.BlockSpec((nb, cout, L), lambda b: (b, 0, 0)),
            cspec, cspec,
        ),
        scratch_shapes=[pltpu.VMEM((cout, 1), jnp.float32),
                        pltpu.VMEM((cout, 1), jnp.float32)],
        compiler_params=_cparams(),
    )(alpha, gamma.reshape(cout, 1), beta.reshape(cout, 1),
      x_flat, wg, b.reshape(cout, 1), m32)


def _fused_call(alpha, gamma, beta, scale, shift, a_prev, res, w, b, m16, *,
                W, HW, nb, add_self_next):
    B, cin, L = a_prev.shape
    cout = w.shape[1]
    wg = _prep_w(w, jnp.bfloat16)
    has_res = res is not None
    steps = B // nb
    bspec = pl.BlockSpec((nb, cin, L), lambda b: (b, 0, 0))
    cspec = pl.BlockSpec((cout, 1), lambda b: (0, 0))
    in_specs = [
        pl.BlockSpec(memory_space=pltpu.MemorySpace.SMEM),
        cspec, cspec,
        cspec, cspec,
        bspec,
    ]
    args = [alpha, gamma.reshape(cout, 1), beta.reshape(cout, 1),
            scale, shift, a_prev]
    if has_res:
        in_specs.append(bspec)
        args.append(res)
    in_specs += [
        pl.BlockSpec((3, cout, 9 * cin), lambda b: (0, 0, 0)),
        cspec,
        pl.BlockSpec((9, 1, L), lambda b: (0, 0, 0)),
    ]
    args += [wg, b.reshape(cout, 1), m16]

    def body(alpha_ref, gamma_ref, beta_ref, scale_ref, shift_ref,
             a_prev_ref, *rest):
        if has_res:
            res_ref = rest[0]
            rest = rest[1:]
        else:
            res_ref = None
        (wg_ref, b_ref, m_ref, cur_ref, a_ref, oscale_ref, oshift_ref,
         s1_sc, s2_sc) = rest
        _fused_kernel(alpha_ref, gamma_ref, beta_ref, scale_ref, shift_ref,
                      a_prev_ref, res_ref, wg_ref, b_ref, m_ref,
                      cur_ref, a_ref, oscale_ref, oshift_ref, s1_sc, s2_sc,
                      W=W, HW=HW, has_res=has_res, nb=nb, steps=steps,
                      n=float(B * L),
                      add_one=1.0 if add_self_next else 0.0)

    out_shape = (
        jax.ShapeDtypeStruct((B, cout, L), jnp.bfloat16),
        jax.ShapeDtypeStruct((B, cout, L), jnp.bfloat16),
        jax.ShapeDtypeStruct((cout, 1), jnp.float32),
        jax.ShapeDtypeStruct((cout, 1), jnp.float32),
    )
    return pl.pallas_call(
        body,
        out_shape=out_shape,
        grid=(steps,),
        in_specs=in_specs,
        out_specs=(
            pl.BlockSpec((nb, cout, L), lambda b: (b, 0, 0)),
            pl.BlockSpec((nb, cout, L), lambda b: (b, 0, 0)),
            cspec, cspec,
        ),
        scratch_shapes=[pltpu.VMEM((cout, 1), jnp.float32),
                        pltpu.VMEM((cout, 1), jnp.float32)],
        compiler_params=_cparams(),
    )(*args)


def _final_call(scale, shift, a, res, *, nb):
    B, cout, L = a.shape
    bspec = pl.BlockSpec((nb, cout, L), lambda b: (b, 0, 0))
    cspec = pl.BlockSpec((cout, 1), lambda b: (0, 0))
    return pl.pallas_call(
        _final_kernel,
        out_shape=jax.ShapeDtypeStruct((B, cout, L), jnp.float32),
        grid=(B // nb,),
        in_specs=[cspec, cspec, bspec, bspec],
        out_specs=bspec,
        compiler_params=_cparams(),
    )(scale, shift, a, res)


def kernel(img,
           w0, b0, alpha0, gamma0, beta0,
           w1, b1, alpha1, gamma1, beta1,
           w2, b2, alpha2, gamma2, beta2,
           w3, b3, alpha3, gamma3, beta3):
    x = _pixel_shuffle_3d(img, 2)
    B, C0, D, H, W = x.shape
    L = D * H * W
    HW = H * W
    x_flat = x.reshape(B, C0, L)

    m32 = jnp.asarray(_hw_masks_np(D, H, W)).reshape(9, 1, L)
    m16 = m32.astype(jnp.bfloat16)

    nbc = 8 if B % 8 == 0 else 1
    nbf = 8 if B % 8 == 0 else nbc
    a, sc, sh = _conv0_call(alpha0, gamma0, beta0, x_flat, w0, b0, m32,
                            W=W, HW=HW, nb=nbc, add_self_next=True)

    res = None
    for (w, b, alpha, gamma, beta) in (
            (w1, b1, alpha1, gamma1, beta1),
            (w2, b2, alpha2, gamma2, beta2),
            (w3, b3, alpha3, gamma3, beta3)):
        cur, a_new, sc, sh = _fused_call(alpha, gamma, beta, sc, sh, a, res,
                                         w, b, m16, W=W, HW=HW, nb=nbc,
                                         add_self_next=False)
        a, res = a_new, cur

    out = _final_call(sc, sh, a, res, nb=nbf)
    cout = out.shape[1]
    return out.reshape(B, cout, D, H, W)
